# initial kernel scaffold (unmeasured)
import jax
import jax.numpy as jnp
from jax import lax
from jax.experimental import pallas as pl
from jax.experimental.pallas import tpu as pltpu


def kernel(
    x,
):
    def body(*refs):
        pass

    out_shape = jax.ShapeDtypeStruct(..., jnp.float32)
    return pl.pallas_call(body, out_shape=out_shape)(...)


# baseline (device time: 818962 ns/iter reference)
import jax
import jax.numpy as jnp
from jax import lax
from jax.experimental import pallas as pl
from jax.experimental.pallas import tpu as pltpu

M = 16384
N = 2048
NOUT = N // 2
CHUNK = 2048
NCHUNK = M // CHUNK


def kernel(x):
    def body(x_ref, out_ref, recv_buf, local_buf, acc_buf,
             send_sem, recv_sem, local_sem, out_sem, credit_sem):
        my_x = lax.axis_index("x")
        my_y = lax.axis_index("y")
        my_z = lax.axis_index("z")
        partner = (1 - my_x, my_y, my_z)

        mycol = my_x * NOUT
        pcol = (1 - my_x) * NOUT

        barrier_sem = pltpu.get_barrier_semaphore()
        pl.semaphore_signal(
            barrier_sem, inc=1,
            device_id=partner, device_id_type=pl.DeviceIdType.MESH,
        )
        pl.semaphore_wait(barrier_sem, 1)

        for c in range(NCHUNK):
            rows = pl.ds(c * CHUNK, CHUNK)

            if c > 0:
                pl.semaphore_wait(credit_sem, 1)

            rdma = pltpu.make_async_remote_copy(
                src_ref=x_ref.at[0, rows, pl.ds(pcol, NOUT)],
                dst_ref=recv_buf,
                send_sem=send_sem,
                recv_sem=recv_sem,
                device_id=partner,
                device_id_type=pl.DeviceIdType.MESH,
            )
            rdma.start()

            cp = pltpu.make_async_copy(
                x_ref.at[0, rows, pl.ds(mycol, NOUT)], local_buf, local_sem,
            )
            cp.start()
            cp.wait()

            rdma.wait()

            acc_buf[:, :] = local_buf[:, :] + recv_buf[:, :]

            if c < NCHUNK - 1:
                pl.semaphore_signal(
                    credit_sem, inc=1,
                    device_id=partner, device_id_type=pl.DeviceIdType.MESH,
                )

            ocp = pltpu.make_async_copy(acc_buf, out_ref.at[rows, :], out_sem)
            ocp.start()
            ocp.wait()

    return pl.pallas_call(
        body,
        out_shape=jax.ShapeDtypeStruct((M, NOUT), jnp.float32),
        in_specs=[pl.BlockSpec(memory_space=pl.ANY)],
        out_specs=pl.BlockSpec(memory_space=pl.ANY),
        scratch_shapes=[
            pltpu.VMEM((CHUNK, NOUT), jnp.float32),
            pltpu.VMEM((CHUNK, NOUT), jnp.float32),
            pltpu.VMEM((CHUNK, NOUT), jnp.float32),
            pltpu.SemaphoreType.DMA,
            pltpu.SemaphoreType.DMA,
            pltpu.SemaphoreType.DMA,
            pltpu.SemaphoreType.DMA,
            pltpu.SemaphoreType.REGULAR,
        ],
        compiler_params=pltpu.CompilerParams(collective_id=0),
    )(x)


# device time: 509817 ns/iter; 1.6064x vs baseline; 1.6064x over previous
import jax
import jax.numpy as jnp
from jax import lax
from jax.experimental import pallas as pl
from jax.experimental.pallas import tpu as pltpu

M = 16384
N = 2048
NOUT = N // 2
P = 8
BLK = M // P
NR = 4
NL = 3

_MESH = pl.DeviceIdType.MESH


def _ypos(q):
    return jnp.where(q < 4, 0, 1)


def _zpos(q):
    return jnp.where(q < 4, q, 7 - q)


def kernel(x):
    def body(x_ref, out_ref, pbuf, local_buf, own_buf,
             p1_send, p1_recv, local_sem, out_sem,
             rs_sems, rr_sems, ls_sems, lr_sems):
        my_x = lax.axis_index("x")
        my_y = lax.axis_index("y")
        my_z = lax.axis_index("z")
        partner = (1 - my_x, my_y, my_z)
        mycol = my_x * NOUT
        pcol = (1 - my_x) * NOUT

        p = jnp.where(my_y == 0, my_z, 7 - my_z)
        rpos = (p + 1) % P
        lpos = (p + P - 1) % P
        right = (my_x, _ypos(rpos), _zpos(rpos))
        left = (my_x, _ypos(lpos), _zpos(lpos))

        barrier = pltpu.get_barrier_semaphore()
        for nbr in (partner, left, right):
            pl.semaphore_signal(barrier, inc=1, device_id=nbr,
                                device_id_type=_MESH)
        pl.semaphore_wait(barrier, 3)

        def rows(q):
            return pl.ds(q * BLK, BLK)

        ph1 = pltpu.make_async_remote_copy(
            src_ref=x_ref.at[0, rows(p), pl.ds(pcol, NOUT)],
            dst_ref=pbuf,
            send_sem=p1_send,
            recv_sem=p1_recv,
            device_id=partner,
            device_id_type=_MESH,
        )
        ph1.start()
        lcp = pltpu.make_async_copy(
            x_ref.at[0, rows(p), pl.ds(mycol, NOUT)], local_buf, local_sem,
        )
        lcp.start()
        lcp.wait()
        ph1.wait()

        own_buf[:, :] = local_buf[:, :] + pbuf[:, :]
        ocp = pltpu.make_async_copy(own_buf, out_ref.at[rows(p), :], out_sem)
        ocp.start()

        for s in range(NR):
            qr = (p - s + P) % P
            src_r = own_buf if s == 0 else out_ref.at[rows(qr), :]
            rsend = pltpu.make_async_remote_copy(
                src_ref=src_r,
                dst_ref=out_ref.at[rows(qr), :],
                send_sem=rs_sems.at[s],
                recv_sem=rr_sems.at[s],
                device_id=right,
                device_id_type=_MESH,
            )
            rsend.start()

            if s < NL:
                ql = (p + s) % P
                src_l = own_buf if s == 0 else out_ref.at[rows(ql), :]
                lsend = pltpu.make_async_remote_copy(
                    src_ref=src_l,
                    dst_ref=out_ref.at[rows(ql), :],
                    send_sem=ls_sems.at[s],
                    recv_sem=lr_sems.at[s],
                    device_id=left,
                    device_id_type=_MESH,
                )
                lsend.start()

            qin_r = (p - 1 - s + P) % P
            pltpu.make_async_remote_copy(
                src_ref=own_buf,
                dst_ref=out_ref.at[rows(qin_r), :],
                send_sem=rs_sems.at[s],
                recv_sem=rr_sems.at[s],
                device_id=right,
                device_id_type=_MESH,
            ).wait_recv()
            if s < NL:
                qin_l = (p + 1 + s) % P
                pltpu.make_async_remote_copy(
                    src_ref=own_buf,
                    dst_ref=out_ref.at[rows(qin_l), :],
                    send_sem=ls_sems.at[s],
                    recv_sem=lr_sems.at[s],
                    device_id=left,
                    device_id_type=_MESH,
                ).wait_recv()

        for s in range(NR):
            qr = (p - s + P) % P
            src_r = own_buf if s == 0 else out_ref.at[rows(qr), :]
            pltpu.make_async_remote_copy(
                src_ref=src_r,
                dst_ref=out_ref.at[rows(qr), :],
                send_sem=rs_sems.at[s],
                recv_sem=rr_sems.at[s],
                device_id=right,
                device_id_type=_MESH,
            ).wait_send()
        for s in range(NL):
            ql = (p + s) % P
            src_l = own_buf if s == 0 else out_ref.at[rows(ql), :]
            pltpu.make_async_remote_copy(
                src_ref=src_l,
                dst_ref=out_ref.at[rows(ql), :],
                send_sem=ls_sems.at[s],
                recv_sem=lr_sems.at[s],
                device_id=left,
                device_id_type=_MESH,
            ).wait_send()
        ocp.wait()

    return pl.pallas_call(
        body,
        out_shape=jax.ShapeDtypeStruct((M, NOUT), jnp.float32),
        in_specs=[pl.BlockSpec(memory_space=pl.ANY)],
        out_specs=pl.BlockSpec(memory_space=pl.ANY),
        scratch_shapes=[
            pltpu.VMEM((BLK, NOUT), jnp.float32),
            pltpu.VMEM((BLK, NOUT), jnp.float32),
            pltpu.VMEM((BLK, NOUT), jnp.float32),
            pltpu.SemaphoreType.DMA,
            pltpu.SemaphoreType.DMA,
            pltpu.SemaphoreType.DMA,
            pltpu.SemaphoreType.DMA,
            pltpu.SemaphoreType.DMA((NR,)),
            pltpu.SemaphoreType.DMA((NR,)),
            pltpu.SemaphoreType.DMA((NL,)),
            pltpu.SemaphoreType.DMA((NL,)),
        ],
        compiler_params=pltpu.CompilerParams(collective_id=0),
    )(x)


# device time: 435708 ns/iter; 1.8796x vs baseline; 1.1701x over previous
import jax
import jax.numpy as jnp
from jax import lax
from jax.experimental import pallas as pl
from jax.experimental.pallas import tpu as pltpu

M = 16384
N = 2048
NOUT = N // 2
P = 8
BLK = M // P
NR = 4
NL = 3
K = 4
SUB = BLK // K

_MESH = pl.DeviceIdType.MESH


def _ypos(q):
    return jnp.where(q < 4, 0, 1)


def _zpos(q):
    return jnp.where(q < 4, q, 7 - q)


def kernel(x):
    def body(x_ref, out_ref, pbuf, local_buf, own_buf,
             p1_send, p1_recv, local_sem, out_sem,
             rs_sems, rr_sems, ls_sems, lr_sems):
        my_x = lax.axis_index("x")
        my_y = lax.axis_index("y")
        my_z = lax.axis_index("z")
        partner = (1 - my_x, my_y, my_z)
        mycol = my_x * NOUT
        pcol = (1 - my_x) * NOUT

        p = jnp.where(my_y == 0, my_z, 7 - my_z)
        rpos = (p + 1) % P
        lpos = (p + P - 1) % P
        right = (my_x, _ypos(rpos), _zpos(rpos))
        left = (my_x, _ypos(lpos), _zpos(lpos))

        barrier = pltpu.get_barrier_semaphore()
        for nbr in (partner, left, right):
            pl.semaphore_signal(barrier, inc=1, device_id=nbr,
                                device_id_type=_MESH)
        pl.semaphore_wait(barrier, 3)

        def subrows(q, j):
            return pl.ds(q * BLK + j * SUB, SUB)

        for j in range(K):
            pltpu.make_async_remote_copy(
                src_ref=x_ref.at[0, subrows(p, j), pl.ds(pcol, NOUT)],
                dst_ref=pbuf.at[j],
                send_sem=p1_send.at[j],
                recv_sem=p1_recv.at[j],
                device_id=partner,
                device_id_type=_MESH,
            ).start()
        lcp = pltpu.make_async_copy(
            x_ref.at[0, pl.ds(p * BLK, BLK), pl.ds(mycol, NOUT)],
            local_buf, local_sem,
        )
        lcp.start()
        lcp.wait()

        def p1_recv_desc(j):
            return pltpu.make_async_remote_copy(
                src_ref=x_ref.at[0, subrows(p, j), pl.ds(pcol, NOUT)],
                dst_ref=pbuf.at[j],
                send_sem=p1_send.at[j],
                recv_sem=p1_recv.at[j],
                device_id=partner,
                device_id_type=_MESH,
            )

        def send_desc(s, j, direction):
            if direction == "r":
                q = (p - s + P) % P
                sems, rems, tgt = rs_sems, rr_sems, right
            else:
                q = (p + s) % P
                sems, rems, tgt = ls_sems, lr_sems, left
            src = (own_buf.at[pl.ds(j * SUB, SUB), :] if s == 0
                   else out_ref.at[subrows(q, j), :])
            return pltpu.make_async_remote_copy(
                src_ref=src,
                dst_ref=out_ref.at[subrows(q, j), :],
                send_sem=sems.at[s, j],
                recv_sem=rems.at[s, j],
                device_id=tgt,
                device_id_type=_MESH,
            )

        def recv_desc(s, j, direction):
            if direction == "r":
                q = (p - 1 - s + P) % P
                rems, tgt = rr_sems, right
            else:
                q = (p + 1 + s) % P
                rems, tgt = lr_sems, left
            return pltpu.make_async_remote_copy(
                src_ref=own_buf.at[pl.ds(j * SUB, SUB), :],
                dst_ref=out_ref.at[subrows(q, j), :],
                send_sem=rs_sems.at[0, j],
                recv_sem=rems.at[s, j],
                device_id=tgt,
                device_id_type=_MESH,
            )

        for j in range(K):
            p1_recv_desc(j).wait_recv()
            own_buf[pl.ds(j * SUB, SUB), :] = (
                local_buf[pl.ds(j * SUB, SUB), :] + pbuf[j, :, :]
            )
            send_desc(0, j, "r").start()
            send_desc(0, j, "l").start()

        ocp = pltpu.make_async_copy(
            own_buf, out_ref.at[pl.ds(p * BLK, BLK), :], out_sem,
        )
        ocp.start()

        for s in range(1, NR):
            for j in range(K):
                recv_desc(s - 1, j, "r").wait_recv()
                send_desc(s, j, "r").start()
                if s < NL:
                    recv_desc(s - 1, j, "l").wait_recv()
                    send_desc(s, j, "l").start()

        for j in range(K):
            recv_desc(NR - 1, j, "r").wait_recv()
            recv_desc(NL - 1, j, "l").wait_recv()

        for j in range(K):
            p1_recv_desc(j).wait_send()
        for s in range(NR):
            for j in range(K):
                send_desc(s, j, "r").wait_send()
        for s in range(NL):
            for j in range(K):
                send_desc(s, j, "l").wait_send()
        ocp.wait()

    return pl.pallas_call(
        body,
        out_shape=jax.ShapeDtypeStruct((M, NOUT), jnp.float32),
        in_specs=[pl.BlockSpec(memory_space=pl.ANY)],
        out_specs=pl.BlockSpec(memory_space=pl.ANY),
        scratch_shapes=[
            pltpu.VMEM((K, SUB, NOUT), jnp.float32),
            pltpu.VMEM((BLK, NOUT), jnp.float32),
            pltpu.VMEM((BLK, NOUT), jnp.float32),
            pltpu.SemaphoreType.DMA((K,)),
            pltpu.SemaphoreType.DMA((K,)),
            pltpu.SemaphoreType.DMA,
            pltpu.SemaphoreType.DMA,
            pltpu.SemaphoreType.DMA((NR, K)),
            pltpu.SemaphoreType.DMA((NR, K)),
            pltpu.SemaphoreType.DMA((NL, K)),
            pltpu.SemaphoreType.DMA((NL, K)),
        ],
        compiler_params=pltpu.CompilerParams(collective_id=0),
    )(x)


# device time: 389758 ns/iter; 2.1012x vs baseline; 1.1179x over previous
import jax
import jax.numpy as jnp
from jax import lax
from jax.experimental import pallas as pl
from jax.experimental.pallas import tpu as pltpu

M = 16384
N = 2048
NOUT = N // 2
P = 8
BLK = M // P
NR = 4
NL = 4
K = 4
SUB = BLK // K


def _active(s, j, direction):
    if s < 3:
        return True
    return j < 2 if direction == "r" else j >= 2

_MESH = pl.DeviceIdType.MESH


def _ypos(q):
    return jnp.where(q < 4, 0, 1)


def _zpos(q):
    return jnp.where(q < 4, q, 7 - q)


def kernel(x):
    def body(x_ref, out_ref, pbuf, local_buf, own_buf,
             p1_send, p1_recv, local_sem, out_sem,
             rs_sems, rr_sems, ls_sems, lr_sems):
        my_x = lax.axis_index("x")
        my_y = lax.axis_index("y")
        my_z = lax.axis_index("z")
        partner = (1 - my_x, my_y, my_z)
        mycol = my_x * NOUT
        pcol = (1 - my_x) * NOUT

        p = jnp.where(my_y == 0, my_z, 7 - my_z)
        rpos = (p + 1) % P
        lpos = (p + P - 1) % P
        right = (my_x, _ypos(rpos), _zpos(rpos))
        left = (my_x, _ypos(lpos), _zpos(lpos))

        barrier = pltpu.get_barrier_semaphore()
        for nbr in (partner, left, right):
            pl.semaphore_signal(barrier, inc=1, device_id=nbr,
                                device_id_type=_MESH)
        pl.semaphore_wait(barrier, 3)

        def subrows(q, j):
            return pl.ds(q * BLK + j * SUB, SUB)

        for j in range(K):
            pltpu.make_async_remote_copy(
                src_ref=x_ref.at[0, subrows(p, j), pl.ds(pcol, NOUT)],
                dst_ref=pbuf.at[j],
                send_sem=p1_send.at[j],
                recv_sem=p1_recv.at[j],
                device_id=partner,
                device_id_type=_MESH,
            ).start()
        lcp = pltpu.make_async_copy(
            x_ref.at[0, pl.ds(p * BLK, BLK), pl.ds(mycol, NOUT)],
            local_buf, local_sem,
        )
        lcp.start()
        lcp.wait()

        def p1_recv_desc(j):
            return pltpu.make_async_remote_copy(
                src_ref=x_ref.at[0, subrows(p, j), pl.ds(pcol, NOUT)],
                dst_ref=pbuf.at[j],
                send_sem=p1_send.at[j],
                recv_sem=p1_recv.at[j],
                device_id=partner,
                device_id_type=_MESH,
            )

        def send_desc(s, j, direction):
            if direction == "r":
                q = (p - s + P) % P
                sems, rems, tgt = rs_sems, rr_sems, right
            else:
                q = (p + s) % P
                sems, rems, tgt = ls_sems, lr_sems, left
            src = (own_buf.at[pl.ds(j * SUB, SUB), :] if s == 0
                   else out_ref.at[subrows(q, j), :])
            return pltpu.make_async_remote_copy(
                src_ref=src,
                dst_ref=out_ref.at[subrows(q, j), :],
                send_sem=sems.at[s, j],
                recv_sem=rems.at[s, j],
                device_id=tgt,
                device_id_type=_MESH,
            )

        def recv_desc(s, j, direction):
            if direction == "r":
                q = (p - 1 - s + P) % P
                rems, tgt = rr_sems, right
            else:
                q = (p + 1 + s) % P
                rems, tgt = lr_sems, left
            return pltpu.make_async_remote_copy(
                src_ref=own_buf.at[pl.ds(j * SUB, SUB), :],
                dst_ref=out_ref.at[subrows(q, j), :],
                send_sem=rs_sems.at[0, j],
                recv_sem=rems.at[s, j],
                device_id=tgt,
                device_id_type=_MESH,
            )

        for j in range(K):
            p1_recv_desc(j).wait_recv()
            own_buf[pl.ds(j * SUB, SUB), :] = (
                local_buf[pl.ds(j * SUB, SUB), :] + pbuf[j, :, :]
            )
            send_desc(0, j, "r").start()
            send_desc(0, j, "l").start()

        ocp = pltpu.make_async_copy(
            own_buf, out_ref.at[pl.ds(p * BLK, BLK), :], out_sem,
        )
        ocp.start()

        for s in range(1, NR):
            for j in range(K):
                recv_desc(s - 1, j, "r").wait_recv()
                if _active(s, j, "r"):
                    send_desc(s, j, "r").start()
                recv_desc(s - 1, j, "l").wait_recv()
                if _active(s, j, "l"):
                    send_desc(s, j, "l").start()

        for j in range(K):
            if _active(NR - 1, j, "r"):
                recv_desc(NR - 1, j, "r").wait_recv()
            if _active(NL - 1, j, "l"):
                recv_desc(NL - 1, j, "l").wait_recv()

        for j in range(K):
            p1_recv_desc(j).wait_send()
        for s in range(NR):
            for j in range(K):
                if _active(s, j, "r"):
                    send_desc(s, j, "r").wait_send()
        for s in range(NL):
            for j in range(K):
                if _active(s, j, "l"):
                    send_desc(s, j, "l").wait_send()
        ocp.wait()

    return pl.pallas_call(
        body,
        out_shape=jax.ShapeDtypeStruct((M, NOUT), jnp.float32),
        in_specs=[pl.BlockSpec(memory_space=pl.ANY)],
        out_specs=pl.BlockSpec(memory_space=pl.ANY),
        scratch_shapes=[
            pltpu.VMEM((K, SUB, NOUT), jnp.float32),
            pltpu.VMEM((BLK, NOUT), jnp.float32),
            pltpu.VMEM((BLK, NOUT), jnp.float32),
            pltpu.SemaphoreType.DMA((K,)),
            pltpu.SemaphoreType.DMA((K,)),
            pltpu.SemaphoreType.DMA,
            pltpu.SemaphoreType.DMA,
            pltpu.SemaphoreType.DMA((NR, K)),
            pltpu.SemaphoreType.DMA((NR, K)),
            pltpu.SemaphoreType.DMA((NL, K)),
            pltpu.SemaphoreType.DMA((NL, K)),
        ],
        compiler_params=pltpu.CompilerParams(collective_id=0),
    )(x)


# device time: 380243 ns/iter; 2.1538x vs baseline; 1.0250x over previous
import jax
import jax.numpy as jnp
from jax import lax
from jax.experimental import pallas as pl
from jax.experimental.pallas import tpu as pltpu

M = 16384
N = 2048
NOUT = N // 2
P = 8
BLK = M // P
NR = 4
NL = 4
K = 8
SUB = BLK // K


def _active(s, j, direction):
    if s < 3:
        return True
    return j < K // 2 if direction == "r" else j >= K // 2

_MESH = pl.DeviceIdType.MESH


def _ypos(q):
    return jnp.where(q < 4, 0, 1)


def _zpos(q):
    return jnp.where(q < 4, q, 7 - q)


def kernel(x):
    def body(x_ref, out_ref, pbuf, local_buf, own_buf,
             p1_send, p1_recv, local_sem, out_sem,
             rs_sems, rr_sems, ls_sems, lr_sems):
        my_x = lax.axis_index("x")
        my_y = lax.axis_index("y")
        my_z = lax.axis_index("z")
        partner = (1 - my_x, my_y, my_z)
        mycol = my_x * NOUT
        pcol = (1 - my_x) * NOUT

        p = jnp.where(my_y == 0, my_z, 7 - my_z)
        rpos = (p + 1) % P
        lpos = (p + P - 1) % P
        right = (my_x, _ypos(rpos), _zpos(rpos))
        left = (my_x, _ypos(lpos), _zpos(lpos))

        barrier = pltpu.get_barrier_semaphore()
        for nbr in (partner, left, right):
            pl.semaphore_signal(barrier, inc=1, device_id=nbr,
                                device_id_type=_MESH)
        pl.semaphore_wait(barrier, 3)

        def subrows(q, j):
            return pl.ds(q * BLK + j * SUB, SUB)

        for j in range(K):
            pltpu.make_async_remote_copy(
                src_ref=x_ref.at[0, subrows(p, j), pl.ds(pcol, NOUT)],
                dst_ref=pbuf.at[j],
                send_sem=p1_send.at[j],
                recv_sem=p1_recv.at[j],
                device_id=partner,
                device_id_type=_MESH,
            ).start()
        lcp = pltpu.make_async_copy(
            x_ref.at[0, pl.ds(p * BLK, BLK), pl.ds(mycol, NOUT)],
            local_buf, local_sem,
        )
        lcp.start()
        lcp.wait()

        def p1_recv_desc(j):
            return pltpu.make_async_remote_copy(
                src_ref=x_ref.at[0, subrows(p, j), pl.ds(pcol, NOUT)],
                dst_ref=pbuf.at[j],
                send_sem=p1_send.at[j],
                recv_sem=p1_recv.at[j],
                device_id=partner,
                device_id_type=_MESH,
            )

        def send_desc(s, j, direction):
            if direction == "r":
                q = (p - s + P) % P
                sems, rems, tgt = rs_sems, rr_sems, right
            else:
                q = (p + s) % P
                sems, rems, tgt = ls_sems, lr_sems, left
            src = (own_buf.at[pl.ds(j * SUB, SUB), :] if s == 0
                   else out_ref.at[subrows(q, j), :])
            return pltpu.make_async_remote_copy(
                src_ref=src,
                dst_ref=out_ref.at[subrows(q, j), :],
                send_sem=sems.at[s, j],
                recv_sem=rems.at[s, j],
                device_id=tgt,
                device_id_type=_MESH,
            )

        def recv_desc(s, j, direction):
            if direction == "r":
                q = (p - 1 - s + P) % P
                rems, tgt = rr_sems, right
            else:
                q = (p + 1 + s) % P
                rems, tgt = lr_sems, left
            return pltpu.make_async_remote_copy(
                src_ref=own_buf.at[pl.ds(j * SUB, SUB), :],
                dst_ref=out_ref.at[subrows(q, j), :],
                send_sem=rs_sems.at[0, j],
                recv_sem=rems.at[s, j],
                device_id=tgt,
                device_id_type=_MESH,
            )

        for j in range(K):
            p1_recv_desc(j).wait_recv()
            own_buf[pl.ds(j * SUB, SUB), :] = (
                local_buf[pl.ds(j * SUB, SUB), :] + pbuf[j, :, :]
            )
            send_desc(0, j, "r").start()
            send_desc(0, j, "l").start()

        ocp = pltpu.make_async_copy(
            own_buf, out_ref.at[pl.ds(p * BLK, BLK), :], out_sem,
        )
        ocp.start()

        for s in range(1, NR):
            for j in range(K):
                recv_desc(s - 1, j, "r").wait_recv()
                if _active(s, j, "r"):
                    send_desc(s, j, "r").start()
                recv_desc(s - 1, j, "l").wait_recv()
                if _active(s, j, "l"):
                    send_desc(s, j, "l").start()

        for j in range(K):
            if _active(NR - 1, j, "r"):
                recv_desc(NR - 1, j, "r").wait_recv()
            if _active(NL - 1, j, "l"):
                recv_desc(NL - 1, j, "l").wait_recv()

        for j in range(K):
            p1_recv_desc(j).wait_send()
        for s in range(NR):
            for j in range(K):
                if _active(s, j, "r"):
                    send_desc(s, j, "r").wait_send()
        for s in range(NL):
            for j in range(K):
                if _active(s, j, "l"):
                    send_desc(s, j, "l").wait_send()
        ocp.wait()

    return pl.pallas_call(
        body,
        out_shape=jax.ShapeDtypeStruct((M, NOUT), jnp.float32),
        in_specs=[pl.BlockSpec(memory_space=pl.ANY)],
        out_specs=pl.BlockSpec(memory_space=pl.ANY),
        scratch_shapes=[
            pltpu.VMEM((K, SUB, NOUT), jnp.float32),
            pltpu.VMEM((BLK, NOUT), jnp.float32),
            pltpu.VMEM((BLK, NOUT), jnp.float32),
            pltpu.SemaphoreType.DMA((K,)),
            pltpu.SemaphoreType.DMA((K,)),
            pltpu.SemaphoreType.DMA,
            pltpu.SemaphoreType.DMA,
            pltpu.SemaphoreType.DMA((NR, K)),
            pltpu.SemaphoreType.DMA((NR, K)),
            pltpu.SemaphoreType.DMA((NL, K)),
            pltpu.SemaphoreType.DMA((NL, K)),
        ],
        compiler_params=pltpu.CompilerParams(collective_id=0),
    )(x)
